# Initial kernel scaffold; baseline (speedup 1.0000x reference)
import jax
import jax.numpy as jnp
from jax.experimental import pallas as pl


def kernel(x_per_region, targets, params):
    raise NotImplementedError("write your pallas kernel here")


# speedup vs baseline: 1.2987x; 1.2987x over previous
import numpy as np
import jax
import jax.numpy as jnp
from jax.experimental import pallas as pl
from jax.experimental.pallas import tpu as pltpu

R = 19
D = 768
B = 64
V = 32000
F = 7
TOP_M = 2
K_ACT = 6
RTK = 4
NEG = -1e9
MAXN = 6
H4 = 4 * D
VT = 3200
NT = V // VT
FB = F * B

_NEIGHBORS = [[1, 3, 4], [0, 2, 4, 5], [1, 5, 6], [0, 4, 7, 8], [0, 1, 3, 5, 8, 9],
              [1, 2, 4, 6, 9, 10], [2, 5, 10, 11], [3, 8, 12], [3, 4, 7, 9, 12, 13],
              [4, 5, 8, 10, 13, 14], [5, 6, 9, 11, 14, 15], [6, 10, 15], [7, 8, 13, 16],
              [8, 9, 12, 14, 16, 17], [9, 10, 13, 15, 17, 18], [10, 11, 14, 18],
              [12, 13, 17], [13, 14, 16, 18], [14, 15, 17]]


def _pad_nbrs():
    idx = np.zeros((R, MAXN), np.int32)
    val = np.zeros((R, MAXN), np.int32)
    for r, ns in enumerate(_NEIGHBORS):
        for j, n in enumerate(ns):
            idx[r, j] = n
            val[r, j] = 1
    return idx, val


_NBR_IDX_NP, _NBR_VAL_NP = _pad_nbrs()
_DECAY_MEAN = float(((np.float32(1.0) - np.float32(0.0))
                     + (np.float32(1.0) - np.float32(0.5))
                     + (np.float32(1.0) - np.float32(0.9)))
                    / np.float32(3.0))
_LOGV = float(np.log(float(V)))
_SQRT_D = float(np.sqrt(float(D)))


def _f32dot(a, b):
    return jnp.dot(a, b, preferred_element_type=jnp.float32)


def _b16(x):
    return x.astype(jnp.bfloat16).astype(jnp.float32)


def _rowdot(a, b):
    full = jax.lax.dot_general(a, b, (((1,), (1,)), ((), ())),
                               preferred_element_type=jnp.float32)
    ii = jax.lax.broadcasted_iota(jnp.int32, (B, B), 0)
    jj = jax.lax.broadcasted_iota(jnp.int32, (B, B), 1)
    return jnp.sum(jnp.where(ii == jj, full, 0.0), axis=1, keepdims=True)


def _topm_softmax(x, m, n):
    cols = []
    for i in range(n):
        xi = x[:, i:i + 1]
        cnt = jnp.sum(jnp.where(x > xi, 1.0, 0.0), axis=1, keepdims=True)
        cols.append(jnp.where(cnt < m, xi, NEG))
    xk = jnp.concatenate(cols, axis=1)
    mx = jnp.max(xk, axis=1, keepdims=True)
    e = jnp.exp(xk - mx)
    return e / jnp.sum(e, axis=1, keepdims=True)


def _ff_body(x_ref, w1_ref, b1_ref, w2_ref, b2_ref, ws_ref, bs_ref, wg_ref,
             h_ref, sc_ref):
    x = x_ref[0]
    h = jax.nn.gelu(_f32dot(x, w1_ref[0]) + b1_ref[0])
    h = _f32dot(h, w2_ref[0]) + b2_ref[0]
    s = _DECAY_MEAN * jnp.tanh(_f32dot(h, ws_ref[0]) + bs_ref[0])
    Hr = h + s
    h_ref[0] = Hr
    t = _f32dot(Hr, wg_ref[...])
    sc_ref[...] = jnp.reshape(jnp.sum(t, axis=0, keepdims=True) * (1.0 / B),
                              (1, 1, 1))


def _qkv_body(h_ref, wq_ref, wk_ref, wv_ref, q_ref, k_ref, v_ref):
    Hr = h_ref[0]
    q_ref[0] = _f32dot(Hr, wq_ref[...])
    k_ref[0] = _f32dot(Hr, wk_ref[...])
    v_ref[0] = _f32dot(Hr, wv_ref[...])


def _route_body(nbr_idx_ref, nbr_val_ref, srow_ref, h_ref, q0_ref, k0_ref,
                v0_ref, wgf_ref, bg_ref, wp_ref, bp_ref, wbq_ref, wbk_ref,
                h2_ref, bc_ref, bq_ref, bk_ref):
    r = pl.program_id(0)
    lane = jax.lax.broadcasted_iota(jnp.int32, (1, R), 1)
    g = srow_ref[...] + jnp.where((lane == 0) | (lane == R - 1), 1e9, 0.0)

    def region_gate(ridx):
        gr = jnp.sum(jnp.where(lane == ridx, g, 0.0), axis=1, keepdims=True)
        cnt = jnp.sum(jnp.where(g > gr, 1.0, 0.0), axis=1, keepdims=True)
        return jnp.where(cnt < K_ACT, 1.0, 0.0)

    mr = region_gate(r)
    Hs = h_ref[0] * mr
    q = q0_ref[0] * mr

    atts = []
    for j in range(MAXN):
        idx = nbr_idx_ref[r, j]
        valid = nbr_val_ref[r, j]
        kn = k0_ref[idx] * region_gate(idx)
        aj = _rowdot(q, kn) / _SQRT_D
        atts.append(jnp.where(valid > 0, aj, NEG))
    att = jnp.concatenate(atts, axis=1)
    w = _topm_softmax(att, RTK, MAXN)

    wb = _b16(w)
    msg = jnp.zeros((B, D), jnp.float32)
    for j in range(MAXN):
        idx = nbr_idx_ref[r, j]
        vn = _b16(v0_ref[idx] * region_gate(idx))
        msg = msg + wb[:, j:j + 1] * vn
    H2 = Hs + msg
    h2_ref[0] = H2

    egl = _f32dot(H2, wgf_ref[...]) + bg_ref[...]
    ep = _topm_softmax(egl, TOP_M, F)
    bc = jnp.zeros((B, D), jnp.float32)
    for f in range(F):
        hf = jnp.tanh(_f32dot(H2, wp_ref[f]) + bp_ref[f])
        bc = bc + ep[:, f:f + 1] * hf
    bc_ref[0] = bc
    bq_ref[0] = _f32dot(H2, wbq_ref[...])
    bk_ref[0] = _f32dot(H2, wbk_ref[...])


def _motor_body(h2_ref, wm_ref, bm_ref, wgf_ref, bg_ref, wp_ref, bp_ref,
                hfac_ref, gw_ref):
    motor = _f32dot(h2_ref[0], wm_ref[...]) + bm_ref[...]
    gl = _f32dot(motor, wgf_ref[...]) + bg_ref[...]
    gp = _topm_softmax(gl, TOP_M, F)
    for f in range(F):
        hf = jnp.tanh(_f32dot(motor, wp_ref[f]) + bp_ref[f])
        hfac_ref[f * B:(f + 1) * B, :] = hf
        gw_ref[f * B:(f + 1) * B, :] = gp[:, f:f + 1]


def _logits_body(hfac_ref, wemb_ref, lg_ref, m_ref, s_ref):
    t = pl.program_id(0)
    lt = jax.lax.dot_general(hfac_ref[...], wemb_ref[...],
                             (((1,), (1,)), ((), ())),
                             preferred_element_type=jnp.float32)
    lg_ref[...] = lt
    tm = jnp.max(lt, axis=1, keepdims=True)

    @pl.when(t == 0)
    def _():
        m_ref[...] = tm
        s_ref[...] = jnp.sum(jnp.exp(lt - tm), axis=1, keepdims=True)

    @pl.when(t > 0)
    def _():
        mo = m_ref[...]
        mn = jnp.maximum(mo, tm)
        s_ref[...] = (s_ref[...] * jnp.exp(mo - mn)
                      + jnp.sum(jnp.exp(lt - mn), axis=1, keepdims=True))
        m_ref[...] = mn


def _mix_body(tg_ref, lg_ref, m_ref, s_ref, gw_ref, logp_ref, loss_ref):
    t = pl.program_id(0)
    lf = lg_ref[...] - m_ref[...] - jnp.log(s_ref[...])
    p = jnp.exp(lf) * gw_ref[...]
    pm = p[0:B, :]
    for f in range(1, F):
        pm = pm + p[f * B:(f + 1) * B, :]
    lp = jnp.log(pm + 1e-9)
    logp_ref[...] = lp

    tcol = tg_ref[...] - t * VT
    lane = jax.lax.broadcasted_iota(jnp.int32, (B, VT), 1)
    psum = jnp.sum(jnp.where(lane == tcol, lp, 0.0), axis=1, keepdims=True)
    pick = jnp.sum(psum, axis=0, keepdims=True)

    @pl.when(t == 0)
    def _():
        loss_ref[...] = jnp.zeros((1, 1), jnp.float32)

    loss_ref[...] = loss_ref[...] + pick

    @pl.when(t == NT - 1)
    def _():
        loss_ref[...] = loss_ref[...] * (-1.0 / B)


def _bcast_body(nbr_idx_ref, nbr_val_ref, bq_ref, bk_ref, bc_ref, loss_ref,
                out_ref):
    r = pl.program_id(0)
    bq = bq_ref[0]
    atts = []
    for j in range(MAXN):
        idx = nbr_idx_ref[r, j]
        valid = nbr_val_ref[r, j]
        aj = _rowdot(bq, bk_ref[idx]) / _SQRT_D
        atts.append(jnp.where(valid > 0, aj, NEG))
    att = jnp.concatenate(atts, axis=1)
    w = _topm_softmax(att, RTK, MAXN)
    msg = jnp.zeros((B, D), jnp.float32)
    for j in range(MAXN):
        idx = nbr_idx_ref[r, j]
        msg = msg + w[:, j:j + 1] * bc_ref[idx]
    Sv = jnp.clip(loss_ref[...] / _LOGV, 0.0, 1.0)
    boost = 1.0 + 2.0 * jax.nn.sigmoid((Sv - 0.7) * 8.0)
    out_ref[0] = msg * boost


def _rblk(*dims):
    return pl.BlockSpec((1,) + dims, lambda r, *_: (r,) + (0,) * len(dims))


def _const(shape):
    n = len(shape)
    return pl.BlockSpec(shape, lambda *_: (0,) * n)


def kernel(x_per_region, targets, params):
    p = params
    f32 = jnp.float32
    wg_col = p['wgate'].reshape(D, 1)
    b1r = p['b1'].reshape(R, 1, H4)
    b2r = p['b2'].reshape(R, 1, D)
    bsr = p['bs'].reshape(R, 1, D)
    bg_row = p['bg'].reshape(1, F)
    bm_row = p['bmotor'].reshape(1, D)
    bp3 = p['bp'].reshape(F, 1, D)
    tg = targets.reshape(B, 1).astype(jnp.int32)
    nbr_idx = jnp.asarray(_NBR_IDX_NP)
    nbr_val = jnp.asarray(_NBR_VAL_NP)

    Hfull, scores3 = pl.pallas_call(
        _ff_body,
        grid=(R,),
        in_specs=[_rblk(B, D), _rblk(D, H4), _rblk(1, H4), _rblk(H4, D),
                  _rblk(1, D), _rblk(D, D), _rblk(1, D), _const((D, 1))],
        out_specs=[_rblk(B, D),
                   pl.BlockSpec((1, 1, 1), lambda r, *_: (r, 0, 0))],
        out_shape=[jax.ShapeDtypeStruct((R, B, D), f32),
                   jax.ShapeDtypeStruct((R, 1, 1), f32)],
    )(x_per_region, p['W1'], b1r, p['W2'], b2r, p['Ws'], bsr, wg_col)
    srow = scores3.reshape(1, R)

    q0, k0, v0 = pl.pallas_call(
        _qkv_body,
        grid=(R,),
        in_specs=[_rblk(B, D), _const((D, D)), _const((D, D)), _const((D, D))],
        out_specs=[_rblk(B, D)] * 3,
        out_shape=[jax.ShapeDtypeStruct((R, B, D), f32)] * 3,
    )(Hfull, p['Wq'], p['Wk'], p['Wv'])

    H2, Bc0, bqf, bkf = pl.pallas_call(
        _route_body,
        grid_spec=pltpu.PrefetchScalarGridSpec(
            num_scalar_prefetch=2,
            grid=(R,),
            in_specs=[_const((1, R)), _rblk(B, D), _rblk(B, D),
                      _const((R, B, D)), _const((R, B, D)),
                      _const((D, F)), _const((1, F)),
                      _const((F, D, D)), _const((F, 1, D)),
                      _const((D, D)), _const((D, D))],
            out_specs=[_rblk(B, D)] * 4,
        ),
        out_shape=[jax.ShapeDtypeStruct((R, B, D), f32)] * 4,
    )(nbr_idx, nbr_val, srow, Hfull, q0, k0, v0, p['Wg'], bg_row, p['Wp'],
      bp3, p['Wbq'], p['Wbk'])

    hfac, gw = pl.pallas_call(
        _motor_body,
        grid=(1,),
        in_specs=[pl.BlockSpec((1, B, D), lambda i: (R - 1, 0, 0)),
                  _const((D, D)), _const((1, D)), _const((D, F)),
                  _const((1, F)), _const((F, D, D)), _const((F, 1, D))],
        out_specs=[_const((FB, D)), _const((FB, 1))],
        out_shape=[jax.ShapeDtypeStruct((FB, D), f32),
                   jax.ShapeDtypeStruct((FB, 1), f32)],
    )(H2, p['Wmotor'], bm_row, p['Wg'], bg_row, p['Wp'], bp3)

    logits, mrow, srow_sum = pl.pallas_call(
        _logits_body,
        grid=(NT,),
        in_specs=[_const((FB, D)),
                  pl.BlockSpec((VT, D), lambda t: (t, 0))],
        out_specs=[pl.BlockSpec((FB, VT), lambda t: (0, t)),
                   _const((FB, 1)), _const((FB, 1))],
        out_shape=[jax.ShapeDtypeStruct((FB, V), f32),
                   jax.ShapeDtypeStruct((FB, 1), f32),
                   jax.ShapeDtypeStruct((FB, 1), f32)],
    )(hfac, p['Wemb'])

    logp, loss2 = pl.pallas_call(
        _mix_body,
        grid=(NT,),
        in_specs=[_const((B, 1)),
                  pl.BlockSpec((FB, VT), lambda t: (0, t)),
                  _const((FB, 1)), _const((FB, 1)), _const((FB, 1))],
        out_specs=[pl.BlockSpec((B, VT), lambda t: (0, t)),
                   _const((1, 1))],
        out_shape=[jax.ShapeDtypeStruct((B, V), f32),
                   jax.ShapeDtypeStruct((1, 1), f32)],
    )(tg, logits, mrow, srow_sum, gw)

    msg_next = pl.pallas_call(
        _bcast_body,
        grid_spec=pltpu.PrefetchScalarGridSpec(
            num_scalar_prefetch=2,
            grid=(R,),
            in_specs=[_rblk(B, D), _const((R, B, D)), _const((R, B, D)),
                      _const((1, 1))],
            out_specs=[_rblk(B, D)],
        ),
        out_shape=[jax.ShapeDtypeStruct((R, B, D), f32)],
    )(nbr_idx, nbr_val, bqf, bkf, Bc0, loss2)[0]

    return logp, loss2.reshape(()), msg_next


# speedup vs baseline: 1.4491x; 1.1158x over previous
import numpy as np
import jax
import jax.numpy as jnp
from jax.experimental import pallas as pl
from jax.experimental.pallas import tpu as pltpu

R = 19
D = 768
B = 64
V = 32000
F = 7
TOP_M = 2
K_ACT = 6
RTK = 4
NEG = -1e9
MAXN = 6
H4 = 4 * D
VT = 3200
NT = V // VT
FB = F * B

_NEIGHBORS = [[1, 3, 4], [0, 2, 4, 5], [1, 5, 6], [0, 4, 7, 8], [0, 1, 3, 5, 8, 9],
              [1, 2, 4, 6, 9, 10], [2, 5, 10, 11], [3, 8, 12], [3, 4, 7, 9, 12, 13],
              [4, 5, 8, 10, 13, 14], [5, 6, 9, 11, 14, 15], [6, 10, 15], [7, 8, 13, 16],
              [8, 9, 12, 14, 16, 17], [9, 10, 13, 15, 17, 18], [10, 11, 14, 18],
              [12, 13, 17], [13, 14, 16, 18], [14, 15, 17]]


def _pad_nbrs():
    idx = np.zeros((R, MAXN), np.int32)
    val = np.zeros((R, MAXN), np.int32)
    for r, ns in enumerate(_NEIGHBORS):
        for j, n in enumerate(ns):
            idx[r, j] = n
            val[r, j] = 1
    return idx, val


_NBR_IDX_NP, _NBR_VAL_NP = _pad_nbrs()
_DECAY_MEAN = float(((np.float32(1.0) - np.float32(0.0))
                     + (np.float32(1.0) - np.float32(0.5))
                     + (np.float32(1.0) - np.float32(0.9)))
                    / np.float32(3.0))
_LOGV = float(np.log(float(V)))
_SQRT_D = float(np.sqrt(float(D)))


def _f32dot(a, b):
    return jnp.dot(a, b, preferred_element_type=jnp.float32)


def _b16(x):
    return x.astype(jnp.bfloat16).astype(jnp.float32)


def _rowdot(a, b):
    full = jax.lax.dot_general(a, b, (((1,), (1,)), ((), ())),
                               preferred_element_type=jnp.float32)
    ii = jax.lax.broadcasted_iota(jnp.int32, (B, B), 0)
    jj = jax.lax.broadcasted_iota(jnp.int32, (B, B), 1)
    return jnp.sum(jnp.where(ii == jj, full, 0.0), axis=1, keepdims=True)


def _topm_softmax(x, m, n):
    cols = []
    for i in range(n):
        xi = x[:, i:i + 1]
        cnt = jnp.sum(jnp.where(x > xi, 1.0, 0.0), axis=1, keepdims=True)
        cols.append(jnp.where(cnt < m, xi, NEG))
    xk = jnp.concatenate(cols, axis=1)
    mx = jnp.max(xk, axis=1, keepdims=True)
    e = jnp.exp(xk - mx)
    return e / jnp.sum(e, axis=1, keepdims=True)


def _ff_body(x_ref, w1_ref, b1_ref, w2_ref, b2_ref, ws_ref, bs_ref, wg_ref,
             wq_ref, wk_ref, wv_ref, h_ref, sc_ref, q_ref, k_ref, v_ref):
    x = x_ref[0]
    h = jax.nn.gelu(_f32dot(x, w1_ref[0]) + b1_ref[0])
    h = _f32dot(h, w2_ref[0]) + b2_ref[0]
    s = _DECAY_MEAN * jnp.tanh(_f32dot(h, ws_ref[0]) + bs_ref[0])
    Hr = h + s
    h_ref[0] = Hr
    t = _f32dot(Hr, wg_ref[...])
    sc_ref[...] = jnp.reshape(jnp.sum(t, axis=0, keepdims=True) * (1.0 / B),
                              (1, 1, 1))
    q_ref[0] = _f32dot(Hr, wq_ref[...])
    k_ref[0] = _f32dot(Hr, wk_ref[...])
    v_ref[0] = _f32dot(Hr, wv_ref[...])


def _route_body(nbr_idx_ref, nbr_val_ref, srow_ref, h_ref, q0_ref, k0_ref,
                v0_ref, wgf_ref, bg_ref, wp_ref, bp_ref, wbq_ref, wbk_ref,
                wm_ref, bm_ref,
                bc_ref, bq_ref, bk_ref, hfac_ref, gw_ref):
    r = pl.program_id(0)
    lane = jax.lax.broadcasted_iota(jnp.int32, (1, R), 1)
    g = srow_ref[...] + jnp.where((lane == 0) | (lane == R - 1), 1e9, 0.0)

    def region_gate(ridx):
        gr = jnp.sum(jnp.where(lane == ridx, g, 0.0), axis=1, keepdims=True)
        cnt = jnp.sum(jnp.where(g > gr, 1.0, 0.0), axis=1, keepdims=True)
        return jnp.where(cnt < K_ACT, 1.0, 0.0)

    mr = region_gate(r)
    Hs = h_ref[0] * mr
    q = q0_ref[0] * mr

    atts = []
    for j in range(MAXN):
        idx = nbr_idx_ref[r, j]
        valid = nbr_val_ref[r, j]
        kn = k0_ref[idx] * region_gate(idx)
        aj = _rowdot(q, kn) / _SQRT_D
        atts.append(jnp.where(valid > 0, aj, NEG))
    att = jnp.concatenate(atts, axis=1)
    w = _topm_softmax(att, RTK, MAXN)

    wb = _b16(w)
    msg = jnp.zeros((B, D), jnp.float32)
    for j in range(MAXN):
        idx = nbr_idx_ref[r, j]
        vn = _b16(v0_ref[idx] * region_gate(idx))
        msg = msg + wb[:, j:j + 1] * vn
    H2 = Hs + msg

    egl = _f32dot(H2, wgf_ref[...]) + bg_ref[...]
    ep = _topm_softmax(egl, TOP_M, F)
    bc = jnp.zeros((B, D), jnp.float32)
    for f in range(F):
        hf = jnp.tanh(_f32dot(H2, wp_ref[f]) + bp_ref[f])
        bc = bc + ep[:, f:f + 1] * hf
    bc_ref[0] = bc
    bq_ref[0] = _f32dot(H2, wbq_ref[...])
    bk_ref[0] = _f32dot(H2, wbk_ref[...])

    @pl.when(r == R - 1)
    def _():
        motor = _f32dot(H2, wm_ref[...]) + bm_ref[...]
        gl = _f32dot(motor, wgf_ref[...]) + bg_ref[...]
        gp = _topm_softmax(gl, TOP_M, F)
        for f in range(F):
            hf = jnp.tanh(_f32dot(motor, wp_ref[f]) + bp_ref[f])
            hfac_ref[f * B:(f + 1) * B, :] = hf
            gw_ref[f * B:(f + 1) * B, :] = gp[:, f:f + 1]


def _logits_body(hfac_ref, wemb_ref, lg_ref, m_ref, s_ref):
    t = pl.program_id(0)
    lt = jax.lax.dot_general(hfac_ref[...], wemb_ref[...],
                             (((1,), (1,)), ((), ())),
                             preferred_element_type=jnp.float32)
    lg_ref[...] = lt.astype(jnp.bfloat16)
    tm = jnp.max(lt, axis=1, keepdims=True)

    @pl.when(t == 0)
    def _():
        m_ref[...] = tm
        s_ref[...] = jnp.sum(jnp.exp(lt - tm), axis=1, keepdims=True)

    @pl.when(t > 0)
    def _():
        mo = m_ref[...]
        mn = jnp.maximum(mo, tm)
        s_ref[...] = (s_ref[...] * jnp.exp(mo - mn)
                      + jnp.sum(jnp.exp(lt - mn), axis=1, keepdims=True))
        m_ref[...] = mn


def _mix_body(tg_ref, lg_ref, m_ref, s_ref, gw_ref, logp_ref, loss_ref):
    t = pl.program_id(0)
    lf = lg_ref[...].astype(jnp.float32) - m_ref[...] - jnp.log(s_ref[...])
    p = jnp.exp(lf) * gw_ref[...]
    pm = p[0:B, :]
    for f in range(1, F):
        pm = pm + p[f * B:(f + 1) * B, :]
    lp = jnp.log(pm + 1e-9)
    logp_ref[...] = lp

    tcol = tg_ref[...] - t * VT
    lane = jax.lax.broadcasted_iota(jnp.int32, (B, VT), 1)
    psum = jnp.sum(jnp.where(lane == tcol, lp, 0.0), axis=1, keepdims=True)
    pick = jnp.sum(psum, axis=0, keepdims=True)

    @pl.when(t == 0)
    def _():
        loss_ref[...] = jnp.zeros((1, 1), jnp.float32)

    loss_ref[...] = loss_ref[...] + pick

    @pl.when(t == NT - 1)
    def _():
        loss_ref[...] = loss_ref[...] * (-1.0 / B)


def _bcast_body(nbr_idx_ref, nbr_val_ref, bq_ref, bk_ref, bc_ref, loss_ref,
                out_ref):
    r = pl.program_id(0)
    bq = bq_ref[0]
    atts = []
    for j in range(MAXN):
        idx = nbr_idx_ref[r, j]
        valid = nbr_val_ref[r, j]
        aj = _rowdot(bq, bk_ref[idx]) / _SQRT_D
        atts.append(jnp.where(valid > 0, aj, NEG))
    att = jnp.concatenate(atts, axis=1)
    w = _topm_softmax(att, RTK, MAXN)
    msg = jnp.zeros((B, D), jnp.float32)
    for j in range(MAXN):
        idx = nbr_idx_ref[r, j]
        msg = msg + w[:, j:j + 1] * bc_ref[idx]
    Sv = jnp.clip(loss_ref[...] / _LOGV, 0.0, 1.0)
    boost = 1.0 + 2.0 * jax.nn.sigmoid((Sv - 0.7) * 8.0)
    out_ref[0] = msg * boost


def _rblk(*dims):
    return pl.BlockSpec((1,) + dims, lambda r, *_: (r,) + (0,) * len(dims))


def _const(shape):
    n = len(shape)
    return pl.BlockSpec(shape, lambda *_: (0,) * n)


def kernel(x_per_region, targets, params):
    p = params
    f32 = jnp.float32
    wg_col = p['wgate'].reshape(D, 1)
    b1r = p['b1'].reshape(R, 1, H4)
    b2r = p['b2'].reshape(R, 1, D)
    bsr = p['bs'].reshape(R, 1, D)
    bg_row = p['bg'].reshape(1, F)
    bm_row = p['bmotor'].reshape(1, D)
    bp3 = p['bp'].reshape(F, 1, D)
    tg = targets.reshape(B, 1).astype(jnp.int32)
    nbr_idx = jnp.asarray(_NBR_IDX_NP)
    nbr_val = jnp.asarray(_NBR_VAL_NP)

    Hfull, scores3, q0, k0, v0 = pl.pallas_call(
        _ff_body,
        grid=(R,),
        in_specs=[_rblk(B, D), _rblk(D, H4), _rblk(1, H4), _rblk(H4, D),
                  _rblk(1, D), _rblk(D, D), _rblk(1, D), _const((D, 1)),
                  _const((D, D)), _const((D, D)), _const((D, D))],
        out_specs=[_rblk(B, D),
                   pl.BlockSpec((1, 1, 1), lambda r, *_: (r, 0, 0)),
                   _rblk(B, D), _rblk(B, D), _rblk(B, D)],
        out_shape=[jax.ShapeDtypeStruct((R, B, D), f32),
                   jax.ShapeDtypeStruct((R, 1, 1), f32),
                   jax.ShapeDtypeStruct((R, B, D), f32),
                   jax.ShapeDtypeStruct((R, B, D), f32),
                   jax.ShapeDtypeStruct((R, B, D), f32)],
    )(x_per_region, p['W1'], b1r, p['W2'], b2r, p['Ws'], bsr, wg_col,
      p['Wq'], p['Wk'], p['Wv'])
    srow = scores3.reshape(1, R)

    Bc0, bqf, bkf, hfac, gw = pl.pallas_call(
        _route_body,
        grid_spec=pltpu.PrefetchScalarGridSpec(
            num_scalar_prefetch=2,
            grid=(R,),
            in_specs=[_const((1, R)), _rblk(B, D), _rblk(B, D),
                      _const((R, B, D)), _const((R, B, D)),
                      _const((D, F)), _const((1, F)),
                      _const((F, D, D)), _const((F, 1, D)),
                      _const((D, D)), _const((D, D)),
                      _const((D, D)), _const((1, D))],
            out_specs=[_rblk(B, D), _rblk(B, D), _rblk(B, D),
                       _const((FB, D)), _const((FB, 1))],
        ),
        out_shape=[jax.ShapeDtypeStruct((R, B, D), f32),
                   jax.ShapeDtypeStruct((R, B, D), f32),
                   jax.ShapeDtypeStruct((R, B, D), f32),
                   jax.ShapeDtypeStruct((FB, D), f32),
                   jax.ShapeDtypeStruct((FB, 1), f32)],
    )(nbr_idx, nbr_val, srow, Hfull, q0, k0, v0, p['Wg'], bg_row, p['Wp'],
      bp3, p['Wbq'], p['Wbk'], p['Wmotor'], bm_row)

    logits, mrow, srow_sum = pl.pallas_call(
        _logits_body,
        grid=(NT,),
        in_specs=[_const((FB, D)),
                  pl.BlockSpec((VT, D), lambda t: (t, 0))],
        out_specs=[pl.BlockSpec((FB, VT), lambda t: (0, t)),
                   _const((FB, 1)), _const((FB, 1))],
        out_shape=[jax.ShapeDtypeStruct((FB, V), jnp.bfloat16),
                   jax.ShapeDtypeStruct((FB, 1), f32),
                   jax.ShapeDtypeStruct((FB, 1), f32)],
    )(hfac, p['Wemb'])

    logp, loss2 = pl.pallas_call(
        _mix_body,
        grid=(NT,),
        in_specs=[_const((B, 1)),
                  pl.BlockSpec((FB, VT), lambda t: (0, t)),
                  _const((FB, 1)), _const((FB, 1)), _const((FB, 1))],
        out_specs=[pl.BlockSpec((B, VT), lambda t: (0, t)),
                   _const((1, 1))],
        out_shape=[jax.ShapeDtypeStruct((B, V), f32),
                   jax.ShapeDtypeStruct((1, 1), f32)],
    )(tg, logits, mrow, srow_sum, gw)

    msg_next = pl.pallas_call(
        _bcast_body,
        grid_spec=pltpu.PrefetchScalarGridSpec(
            num_scalar_prefetch=2,
            grid=(R,),
            in_specs=[_rblk(B, D), _const((R, B, D)), _const((R, B, D)),
                      _const((1, 1))],
            out_specs=[_rblk(B, D)],
        ),
        out_shape=[jax.ShapeDtypeStruct((R, B, D), f32)],
    )(nbr_idx, nbr_val, bqf, bkf, Bc0, loss2)[0]

    return logp, loss2.reshape(()), msg_next


# speedup vs baseline: 1.4775x; 1.0196x over previous
import numpy as np
import jax
import jax.numpy as jnp
from jax.experimental import pallas as pl
from jax.experimental.pallas import tpu as pltpu

R = 19
D = 768
B = 64
V = 32000
F = 7
TOP_M = 2
K_ACT = 6
RTK = 4
NEG = -1e9
MAXN = 6
H4 = 4 * D
VT = 3200
NT = V // VT
FB = F * B

_NEIGHBORS = [[1, 3, 4], [0, 2, 4, 5], [1, 5, 6], [0, 4, 7, 8], [0, 1, 3, 5, 8, 9],
              [1, 2, 4, 6, 9, 10], [2, 5, 10, 11], [3, 8, 12], [3, 4, 7, 9, 12, 13],
              [4, 5, 8, 10, 13, 14], [5, 6, 9, 11, 14, 15], [6, 10, 15], [7, 8, 13, 16],
              [8, 9, 12, 14, 16, 17], [9, 10, 13, 15, 17, 18], [10, 11, 14, 18],
              [12, 13, 17], [13, 14, 16, 18], [14, 15, 17]]


def _pad_nbrs():
    idx = np.zeros((R, MAXN), np.int32)
    val = np.zeros((R, MAXN), np.int32)
    for r, ns in enumerate(_NEIGHBORS):
        for j, n in enumerate(ns):
            idx[r, j] = n
            val[r, j] = 1
    return idx, val


_NBR_IDX_NP, _NBR_VAL_NP = _pad_nbrs()
_DECAY_MEAN = float(((np.float32(1.0) - np.float32(0.0))
                     + (np.float32(1.0) - np.float32(0.5))
                     + (np.float32(1.0) - np.float32(0.9)))
                    / np.float32(3.0))
_LOGV = float(np.log(float(V)))
_SQRT_D = float(np.sqrt(float(D)))


def _f32dot(a, b):
    return jnp.dot(a, b, preferred_element_type=jnp.float32)


def _b16(x):
    return x.astype(jnp.bfloat16).astype(jnp.float32)


def _rowdot(a, b):
    full = jax.lax.dot_general(a, b, (((1,), (1,)), ((), ())),
                               preferred_element_type=jnp.float32)
    ii = jax.lax.broadcasted_iota(jnp.int32, (B, B), 0)
    jj = jax.lax.broadcasted_iota(jnp.int32, (B, B), 1)
    return jnp.sum(jnp.where(ii == jj, full, 0.0), axis=1, keepdims=True)


def _topm_softmax(x, m, n):
    cols = []
    for i in range(n):
        xi = x[:, i:i + 1]
        cnt = jnp.sum(jnp.where(x > xi, 1.0, 0.0), axis=1, keepdims=True)
        cols.append(jnp.where(cnt < m, xi, NEG))
    xk = jnp.concatenate(cols, axis=1)
    mx = jnp.max(xk, axis=1, keepdims=True)
    e = jnp.exp(xk - mx)
    return e / jnp.sum(e, axis=1, keepdims=True)


def _ff_body(x_ref, w1_ref, b1_ref, w2_ref, b2_ref, ws_ref, bs_ref, wg_ref,
             wq_ref, wk_ref, wv_ref, h_ref, sc_ref, q_ref, k_ref, v_ref):
    x = x_ref[0]
    h = jax.nn.gelu(_f32dot(x, w1_ref[0]) + b1_ref[0])
    h = _f32dot(h, w2_ref[0]) + b2_ref[0]
    s = _DECAY_MEAN * jnp.tanh(_f32dot(h, ws_ref[0]) + bs_ref[0])
    Hr = h + s
    h_ref[0] = Hr
    t = _f32dot(Hr, wg_ref[...])
    sc_ref[...] = jnp.reshape(jnp.sum(t, axis=0, keepdims=True) * (1.0 / B),
                              (1, 1, 1))
    q_ref[0] = _f32dot(Hr, wq_ref[...])
    k_ref[0] = _f32dot(Hr, wk_ref[...])
    v_ref[0] = _f32dot(Hr, wv_ref[...])


def _route_body(nbr_idx_ref, nbr_val_ref, srow_ref, h_ref, q0_ref, k0_ref,
                v0_ref, wgf_ref, bg_ref, wp_ref, bp_ref, wbq_ref, wbk_ref,
                wm_ref, bm_ref,
                bc_ref, bq_ref, bk_ref, hfac_ref, gw_ref):
    r = pl.program_id(0)
    lane = jax.lax.broadcasted_iota(jnp.int32, (1, R), 1)
    g = srow_ref[...] + jnp.where((lane == 0) | (lane == R - 1), 1e9, 0.0)

    def region_gate(ridx):
        gr = jnp.sum(jnp.where(lane == ridx, g, 0.0), axis=1, keepdims=True)
        cnt = jnp.sum(jnp.where(g > gr, 1.0, 0.0), axis=1, keepdims=True)
        return jnp.where(cnt < K_ACT, 1.0, 0.0)

    mr = region_gate(r)
    Hs = h_ref[0] * mr
    q = q0_ref[0] * mr

    atts = []
    for j in range(MAXN):
        idx = nbr_idx_ref[r, j]
        valid = nbr_val_ref[r, j]
        kn = k0_ref[idx] * region_gate(idx)
        aj = _rowdot(q, kn) / _SQRT_D
        atts.append(jnp.where(valid > 0, aj, NEG))
    att = jnp.concatenate(atts, axis=1)
    w = _topm_softmax(att, RTK, MAXN)

    wb = _b16(w)
    msg = jnp.zeros((B, D), jnp.float32)
    for j in range(MAXN):
        idx = nbr_idx_ref[r, j]
        vn = _b16(v0_ref[idx] * region_gate(idx))
        msg = msg + wb[:, j:j + 1] * vn
    H2 = Hs + msg

    egl = _f32dot(H2, wgf_ref[...]) + bg_ref[...]
    ep = _topm_softmax(egl, TOP_M, F)
    bc = jnp.zeros((B, D), jnp.float32)
    for f in range(F):
        hf = jnp.tanh(_f32dot(H2, wp_ref[f]) + bp_ref[f])
        bc = bc + ep[:, f:f + 1] * hf
    bc_ref[0] = bc
    bq_ref[0] = _f32dot(H2, wbq_ref[...])
    bk_ref[0] = _f32dot(H2, wbk_ref[...])

    @pl.when(r == R - 1)
    def _():
        motor = _f32dot(H2, wm_ref[...]) + bm_ref[...]
        gl = _f32dot(motor, wgf_ref[...]) + bg_ref[...]
        gp = _topm_softmax(gl, TOP_M, F)
        for f in range(F):
            hf = jnp.tanh(_f32dot(motor, wp_ref[f]) + bp_ref[f])
            hfac_ref[f * B:(f + 1) * B, :] = hf
            gw_ref[f * B:(f + 1) * B, :] = gp[:, f:f + 1]


def _logits_body(hfac_ref, wemb_ref, lg_ref, m_ref, s_ref):
    t = pl.program_id(0)
    lt = jax.lax.dot_general(hfac_ref[...], wemb_ref[...],
                             (((1,), (1,)), ((), ())),
                             preferred_element_type=jnp.float32)
    lg_ref[...] = lt.astype(jnp.bfloat16)
    tm = jnp.max(lt, axis=1, keepdims=True)

    @pl.when(t == 0)
    def _():
        m_ref[...] = tm
        s_ref[...] = jnp.sum(jnp.exp(lt - tm), axis=1, keepdims=True)

    @pl.when(t > 0)
    def _():
        mo = m_ref[...]
        mn = jnp.maximum(mo, tm)
        s_ref[...] = (s_ref[...] * jnp.exp(mo - mn)
                      + jnp.sum(jnp.exp(lt - mn), axis=1, keepdims=True))
        m_ref[...] = mn


def _mix_body(tg_ref, lg_ref, m_ref, s_ref, gw_ref, bq_ref, bk_ref, bc_ref,
              logp_ref, loss_ref, out_ref):
    t = pl.program_id(0)
    lf = lg_ref[...].astype(jnp.float32) - m_ref[...] - jnp.log(s_ref[...])
    p = jnp.exp(lf) * gw_ref[...]
    pm = p[0:B, :]
    for f in range(1, F):
        pm = pm + p[f * B:(f + 1) * B, :]
    lp = jnp.log(pm + 1e-9)
    logp_ref[...] = lp

    tcol = tg_ref[...] - t * VT
    lane = jax.lax.broadcasted_iota(jnp.int32, (B, VT), 1)
    psum = jnp.sum(jnp.where(lane == tcol, lp, 0.0), axis=1, keepdims=True)
    pick = jnp.sum(psum, axis=0, keepdims=True)

    @pl.when(t == 0)
    def _():
        loss_ref[...] = jnp.zeros((1, 1), jnp.float32)

    loss_ref[...] = loss_ref[...] + pick

    @pl.when(t == NT - 1)
    def _():
        loss_ref[...] = loss_ref[...] * (-1.0 / B)

    @pl.when(t == NT - 1)
    def _():
        Sv = jnp.clip(loss_ref[...] / _LOGV, 0.0, 1.0)
        boost = 1.0 + 2.0 * jax.nn.sigmoid((Sv - 0.7) * 8.0)
        for rr in range(R):
            nbrs = _NEIGHBORS[rr]
            bq = bq_ref[rr]
            atts = [_rowdot(bq, bk_ref[n]) / _SQRT_D for n in nbrs]
            att = jnp.concatenate(atts, axis=1)
            w = _topm_softmax(att, RTK, len(nbrs))
            msg = jnp.zeros((B, D), jnp.float32)
            for j, n in enumerate(nbrs):
                msg = msg + w[:, j:j + 1] * bc_ref[n]
            out_ref[rr] = msg * boost


def _rblk(*dims):
    return pl.BlockSpec((1,) + dims, lambda r, *_: (r,) + (0,) * len(dims))


def _const(shape):
    n = len(shape)
    return pl.BlockSpec(shape, lambda *_: (0,) * n)


def kernel(x_per_region, targets, params):
    p = params
    f32 = jnp.float32
    wg_col = p['wgate'].reshape(D, 1)
    b1r = p['b1'].reshape(R, 1, H4)
    b2r = p['b2'].reshape(R, 1, D)
    bsr = p['bs'].reshape(R, 1, D)
    bg_row = p['bg'].reshape(1, F)
    bm_row = p['bmotor'].reshape(1, D)
    bp3 = p['bp'].reshape(F, 1, D)
    tg = targets.reshape(B, 1).astype(jnp.int32)
    nbr_idx = jnp.asarray(_NBR_IDX_NP)
    nbr_val = jnp.asarray(_NBR_VAL_NP)

    Hfull, scores3, q0, k0, v0 = pl.pallas_call(
        _ff_body,
        grid=(R,),
        in_specs=[_rblk(B, D), _rblk(D, H4), _rblk(1, H4), _rblk(H4, D),
                  _rblk(1, D), _rblk(D, D), _rblk(1, D), _const((D, 1)),
                  _const((D, D)), _const((D, D)), _const((D, D))],
        out_specs=[_rblk(B, D),
                   pl.BlockSpec((1, 1, 1), lambda r, *_: (r, 0, 0)),
                   _rblk(B, D), _rblk(B, D), _rblk(B, D)],
        out_shape=[jax.ShapeDtypeStruct((R, B, D), f32),
                   jax.ShapeDtypeStruct((R, 1, 1), f32),
                   jax.ShapeDtypeStruct((R, B, D), f32),
                   jax.ShapeDtypeStruct((R, B, D), f32),
                   jax.ShapeDtypeStruct((R, B, D), f32)],
    )(x_per_region, p['W1'], b1r, p['W2'], b2r, p['Ws'], bsr, wg_col,
      p['Wq'], p['Wk'], p['Wv'])
    srow = scores3.reshape(1, R)

    Bc0, bqf, bkf, hfac, gw = pl.pallas_call(
        _route_body,
        grid_spec=pltpu.PrefetchScalarGridSpec(
            num_scalar_prefetch=2,
            grid=(R,),
            in_specs=[_const((1, R)), _rblk(B, D), _rblk(B, D),
                      _const((R, B, D)), _const((R, B, D)),
                      _const((D, F)), _const((1, F)),
                      _const((F, D, D)), _const((F, 1, D)),
                      _const((D, D)), _const((D, D)),
                      _const((D, D)), _const((1, D))],
            out_specs=[_rblk(B, D), _rblk(B, D), _rblk(B, D),
                       _const((FB, D)), _const((FB, 1))],
        ),
        out_shape=[jax.ShapeDtypeStruct((R, B, D), f32),
                   jax.ShapeDtypeStruct((R, B, D), f32),
                   jax.ShapeDtypeStruct((R, B, D), f32),
                   jax.ShapeDtypeStruct((FB, D), f32),
                   jax.ShapeDtypeStruct((FB, 1), f32)],
    )(nbr_idx, nbr_val, srow, Hfull, q0, k0, v0, p['Wg'], bg_row, p['Wp'],
      bp3, p['Wbq'], p['Wbk'], p['Wmotor'], bm_row)

    logits, mrow, srow_sum = pl.pallas_call(
        _logits_body,
        grid=(NT,),
        in_specs=[_const((FB, D)),
                  pl.BlockSpec((VT, D), lambda t: (t, 0))],
        out_specs=[pl.BlockSpec((FB, VT), lambda t: (0, t)),
                   _const((FB, 1)), _const((FB, 1))],
        out_shape=[jax.ShapeDtypeStruct((FB, V), jnp.bfloat16),
                   jax.ShapeDtypeStruct((FB, 1), f32),
                   jax.ShapeDtypeStruct((FB, 1), f32)],
    )(hfac, p['Wemb'])

    logp, loss2, msg_next = pl.pallas_call(
        _mix_body,
        grid=(NT,),
        in_specs=[_const((B, 1)),
                  pl.BlockSpec((FB, VT), lambda t: (0, t)),
                  _const((FB, 1)), _const((FB, 1)), _const((FB, 1)),
                  _const((R, B, D)), _const((R, B, D)), _const((R, B, D))],
        out_specs=[pl.BlockSpec((B, VT), lambda t: (0, t)),
                   _const((1, 1)), _const((R, B, D))],
        out_shape=[jax.ShapeDtypeStruct((B, V), f32),
                   jax.ShapeDtypeStruct((1, 1), f32),
                   jax.ShapeDtypeStruct((R, B, D), f32)],
    )(tg, logits, mrow, srow_sum, gw, bqf, bkf, Bc0)

    return logp, loss2.reshape(()), msg_next
